# baseline (speedup 1.0000x reference)
import functools

import jax
import jax.numpy as jnp
from jax import lax
from jax.experimental import pallas as pl
from jax.experimental.pallas import tpu as pltpu
from jax.experimental.pallas import tpu_sc as plsc

GAMMA = 1e-10
K = 10
NUM_CORES = 2
NUM_SUBCORES = 16
LANES = 16


def _sc_diff_kernel(n_rows, row_len):
    nw = NUM_CORES * NUM_SUBCORES
    rows_per_w = n_rows // nw
    n_chunks = rows_per_w // LANES
    half = row_len // 2

    mesh = plsc.VectorSubcoreMesh(core_axis_name="c", subcore_axis_name="s")

    @functools.partial(
        pl.kernel,
        out_type=jax.ShapeDtypeStruct((nw, n_chunks, LANES), jnp.float32),
        mesh=mesh,
        compiler_params=pltpu.CompilerParams(needs_layout_passes=False),
        scratch_types=[
            pltpu.VMEM((LANES, row_len), jnp.float32),
            pltpu.VMEM((LANES, row_len), jnp.float32),
            pltpu.VMEM((LANES, row_len), jnp.float32),
            pltpu.VMEM((LANES, row_len), jnp.float32),
            pltpu.VMEM((n_chunks, LANES), jnp.float32),
            pltpu.SemaphoreType.DMA,
            pltpu.SemaphoreType.DMA,
            pltpu.SemaphoreType.DMA,
            pltpu.SemaphoreType.DMA,
        ],
    )
    def body(s_hbm, t_hbm, d_hbm, t0, s0, t1, s1, d_all, mt0, ms0, mt1, ms1):
        wid = lax.axis_index("s") * NUM_CORES + lax.axis_index("c")
        row0 = wid * rows_per_w
        lane = lax.iota(jnp.int32, LANES)

        def start_in(g, t_buf, s_buf, t_sem, s_sem):
            r = row0 + g * LANES
            pltpu.make_async_copy(
                t_hbm.at[pl.ds(r, LANES), :], t_buf, t_sem).start()
            pltpu.make_async_copy(
                s_hbm.at[pl.ds(r, LANES), :], s_buf, s_sem).start()

        def wait_in(g, t_buf, s_buf, t_sem, s_sem):
            r = row0 + g * LANES
            pltpu.make_async_copy(
                t_hbm.at[pl.ds(r, LANES), :], t_buf, t_sem).wait()
            pltpu.make_async_copy(
                s_hbm.at[pl.ds(r, LANES), :], s_buf, s_sem).wait()

        def compute(g, t_buf, s_buf):
            neg_inf = jnp.full((LANES,), -jnp.inf, jnp.float32)

            @pl.loop(0, half, init_carry=(neg_inf,) * (2 * K), unroll=2)
            def p1(i, carry):
                ta = carry[:K]
                tb = carry[K:]
                ia = jnp.full((LANES,), 0, jnp.int32) + i
                xa = plsc.load_gather(t_buf, [lane, ia])
                xb = plsc.load_gather(t_buf, [lane, ia + half])
                na, nb = [], []
                for j in range(K):
                    na.append(jnp.maximum(ta[j], xa))
                    xa = jnp.minimum(ta[j], xa)
                    nb.append(jnp.maximum(tb[j], xb))
                    xb = jnp.minimum(tb[j], xb)
                return tuple(na) + tuple(nb)

            ta = p1[:K]
            tb = p1[K:]
            m = [jnp.maximum(ta[j], tb[K - 1 - j]) for j in range(K)]
            while len(m) > 1:
                m = [jnp.minimum(m[2 * i], m[2 * i + 1])
                     for i in range(len(m) // 2)] + m[len(m) & ~1:]
            thresh = m[0]

            zeros = jnp.zeros((LANES,), jnp.float32)

            @pl.loop(0, half, init_carry=(zeros,) * 4, unroll=2)
            def p2(i, carry):
                a_all, a_top, b_all, b_top = carry
                ia = jnp.full((LANES,), 0, jnp.int32) + i
                t_a = plsc.load_gather(t_buf, [lane, ia])
                s_a = plsc.load_gather(s_buf, [lane, ia])
                t_b = plsc.load_gather(t_buf, [lane, ia + half])
                s_b = plsc.load_gather(s_buf, [lane, ia + half])
                a_all = a_all + s_a
                a_top = a_top + jnp.where(t_a >= thresh, s_a, jnp.float32(0.0))
                b_all = b_all + s_b
                b_top = b_top + jnp.where(t_b >= thresh, s_b, jnp.float32(0.0))
                return (a_all, a_top, b_all, b_top)

            a_all, a_top, b_all, b_top = p2
            s_all = a_all + b_all
            s_top = a_top + b_top
            d = s_top * jnp.float32(1.0 / K) - (s_all - s_top) * jnp.float32(
                1.0 / (row_len - K)
            )
            d_all[g, :] = d

        start_in(0, t0, s0, mt0, ms0)

        @pl.loop(0, n_chunks // 2)
        def outer(p):
            g0 = 2 * p
            start_in(g0 + 1, t1, s1, mt1, ms1)
            wait_in(g0, t0, s0, mt0, ms0)
            compute(g0, t0, s0)

            @pl.when(p < n_chunks // 2 - 1)
            def _():
                start_in(g0 + 2, t0, s0, mt0, ms0)

            wait_in(g0 + 1, t1, s1, mt1, ms1)
            compute(g0 + 1, t1, s1)

        pltpu.sync_copy(d_all, d_hbm.at[wid])

    return body


def _tc_finish(d):
    def body(x_ref, o_ref):
        x = x_ref[...]
        sig = 1.0 / (1.0 + jnp.exp(-x))
        loss = -jnp.mean(jnp.log(jnp.float32(GAMMA) + sig))
        o_ref[...] = loss.reshape(1, 1)

    out = pl.pallas_call(
        body,
        out_shape=jax.ShapeDtypeStruct((1, 1), jnp.float32),
    )(d)
    return out[0, 0]


def kernel(pred_s, pred_t, k, list_len):
    n_rows, row_len = pred_s.shape
    sc = _sc_diff_kernel(n_rows, row_len)
    d = sc(pred_s, pred_t)
    return _tc_finish(d.reshape(128, n_rows // 128))
